# Initial kernel scaffold; baseline (speedup 1.0000x reference)
import jax
import jax.numpy as jnp
from jax.experimental import pallas as pl


def kernel(x, W_enc, b_enc, W_dec, b_dec):
    raise NotImplementedError("write your pallas kernel here")


# speedup vs baseline: 4.9799x; 4.9799x over previous
import functools

import jax
import jax.numpy as jnp
from jax.experimental import pallas as pl

K = 30


def _enc_body(x_ref, wenc_ref, benc_ref, bdec_ref, out_ref):
    xc = x_ref[...] - bdec_ref[...]
    acc = jax.lax.dot_general(
        xc, wenc_ref[...], (((1,), (1,)), ((), ())),
        preferred_element_type=jnp.float32,
    )
    out_ref[...] = acc + benc_ref[...]


def _thr_body(pre_ref, t_ref):
    w = pre_ref[...]

    def body(_, w):
        m = jnp.max(w, axis=1, keepdims=True)
        return jnp.where(w == m, -jnp.inf, w)

    w = jax.lax.fori_loop(0, K - 1, body, w)
    t_ref[...] = jnp.max(w, axis=1, keepdims=True)


def _dec_body(pre_ref, t_ref, wdec_ref, bdec_ref, sp_ref, rec_ref):
    j = pl.program_id(1)
    pre = pre_ref[...]
    s = jnp.where(pre >= t_ref[...], jnp.maximum(pre, 0.0), 0.0)
    sp_ref[...] = s
    contrib = jax.lax.dot_general(
        s, wdec_ref[...], (((1,), (1,)), ((), ())),
        preferred_element_type=jnp.float32,
    )

    @pl.when(j == 0)
    def _():
        rec_ref[...] = contrib + bdec_ref[...]

    @pl.when(j != 0)
    def _():
        rec_ref[...] += contrib


def kernel(x, W_enc, b_enc, W_dec, b_dec):
    B, D = x.shape
    F = W_enc.shape[0]
    b_enc2 = b_enc.reshape(1, F)
    b_dec2 = b_dec.reshape(1, D)

    BM = min(512, B)
    BN = min(512, F)
    pre = pl.pallas_call(
        _enc_body,
        grid=(B // BM, F // BN),
        in_specs=[
            pl.BlockSpec((BM, D), lambda i, j: (i, 0)),
            pl.BlockSpec((BN, D), lambda i, j: (j, 0)),
            pl.BlockSpec((1, BN), lambda i, j: (0, j)),
            pl.BlockSpec((1, D), lambda i, j: (0, 0)),
        ],
        out_specs=pl.BlockSpec((BM, BN), lambda i, j: (i, j)),
        out_shape=jax.ShapeDtypeStruct((B, F), jnp.float32),
    )(x, W_enc, b_enc2, b_dec2)

    BB = min(256, B)
    thresh = pl.pallas_call(
        _thr_body,
        grid=(B // BB,),
        in_specs=[pl.BlockSpec((BB, F), lambda i: (i, 0))],
        out_specs=pl.BlockSpec((BB, 1), lambda i: (i, 0)),
        out_shape=jax.ShapeDtypeStruct((B, 1), jnp.float32),
    )(pre)

    BM2 = min(512, B)
    BN2 = min(1024, F)
    sparse, recon = pl.pallas_call(
        _dec_body,
        grid=(B // BM2, F // BN2),
        in_specs=[
            pl.BlockSpec((BM2, BN2), lambda i, j: (i, j)),
            pl.BlockSpec((BM2, 1), lambda i, j: (i, 0)),
            pl.BlockSpec((D, BN2), lambda i, j: (0, j)),
            pl.BlockSpec((1, D), lambda i, j: (0, 0)),
        ],
        out_specs=[
            pl.BlockSpec((BM2, BN2), lambda i, j: (i, j)),
            pl.BlockSpec((BM2, D), lambda i, j: (i, 0)),
        ],
        out_shape=[
            jax.ShapeDtypeStruct((B, F), jnp.float32),
            jax.ShapeDtypeStruct((B, D), jnp.float32),
        ],
    )(pre, thresh, W_dec, b_dec2)

    return (recon, sparse)


# speedup vs baseline: 15.6651x; 3.1457x over previous
import jax
import jax.numpy as jnp
from jax.experimental import pallas as pl

K = 30
NCAND = 5
LANE = 128


def _enc_body(x_ref, wenc_ref, benc_ref, bdec_ref, out_ref, *m_refs):
    j = pl.program_id(1)
    xc = x_ref[...] - bdec_ref[...]
    acc = jax.lax.dot_general(
        xc, wenc_ref[...], (((1,), (1,)), ((), ())),
        preferred_element_type=jnp.float32,
    )
    pre = acc + benc_ref[...]
    out_ref[...] = pre

    @pl.when(j == 0)
    def _():
        for r in m_refs:
            r[...] = jnp.full(r.shape, -jnp.inf, dtype=jnp.float32)

    m = [r[...] for r in m_refs]
    bn = pre.shape[1]
    for k in range(bn // LANE):
        s = pre[:, k * LANE:(k + 1) * LANE]
        for lvl in range(NCAND):
            new_m = jnp.maximum(m[lvl], s)
            s = jnp.minimum(m[lvl], s)
            m[lvl] = new_m
    for r, v in zip(m_refs, m):
        r[...] = v


def _nth_largest_multi(arrs, n):
    m = jnp.full((arrs[0].shape[0], 1), jnp.inf, dtype=jnp.float32)

    def body(_, m):
        parts = [jnp.where(a < m, a, -jnp.inf) for a in arrs]
        red = parts[0]
        for p in parts[1:]:
            red = jnp.maximum(red, p)
        return jnp.max(red, axis=1, keepdims=True)

    return jax.lax.fori_loop(0, n, body, m)


def _dec_body(pre_ref, m0, m1, m2, m3, m4, wdec_ref, bdec_ref, sp_ref, rec_ref):
    pre = pre_ref[...]
    cands = [m0[...], m1[...], m2[...], m3[...], m4[...]]
    t_cand = _nth_largest_multi(cands, K)
    cnt = jnp.sum((pre >= t_cand).astype(jnp.float32), axis=1, keepdims=True)
    ok = jnp.all(cnt == float(K))
    t = jax.lax.cond(ok, lambda: t_cand, lambda: _nth_largest_multi([pre], K))
    s = jnp.where(pre >= t, jnp.maximum(pre, 0.0), 0.0)
    sp_ref[...] = s
    rec_ref[...] = jax.lax.dot_general(
        s.astype(jnp.bfloat16), wdec_ref[...], (((1,), (1,)), ((), ())),
        preferred_element_type=jnp.float32,
    ) + bdec_ref[...]


def kernel(x, W_enc, b_enc, W_dec, b_dec):
    B, D = x.shape
    F = W_enc.shape[0]
    b_enc2 = b_enc.reshape(1, F)
    b_dec2 = b_dec.reshape(1, D)
    W_dec16 = W_dec.astype(jnp.bfloat16)

    BM = min(1024, B)
    BN = min(1024, F)
    outs = pl.pallas_call(
        _enc_body,
        grid=(B // BM, F // BN),
        in_specs=[
            pl.BlockSpec((BM, D), lambda i, j: (i, 0)),
            pl.BlockSpec((BN, D), lambda i, j: (j, 0)),
            pl.BlockSpec((1, BN), lambda i, j: (0, j)),
            pl.BlockSpec((1, D), lambda i, j: (0, 0)),
        ],
        out_specs=[pl.BlockSpec((BM, BN), lambda i, j: (i, j))]
        + [pl.BlockSpec((BM, LANE), lambda i, j: (i, 0)) for _ in range(NCAND)],
        out_shape=[jax.ShapeDtypeStruct((B, F), jnp.float32)]
        + [jax.ShapeDtypeStruct((B, LANE), jnp.float32) for _ in range(NCAND)],
    )(x, W_enc, b_enc2, b_dec2)
    pre, ms = outs[0], list(outs[1:])

    BB = min(128, B)
    sparse, recon = pl.pallas_call(
        _dec_body,
        grid=(B // BB,),
        in_specs=[pl.BlockSpec((BB, F), lambda i: (i, 0))]
        + [pl.BlockSpec((BB, LANE), lambda i: (i, 0)) for _ in range(NCAND)]
        + [
            pl.BlockSpec((D, F), lambda i: (0, 0)),
            pl.BlockSpec((1, D), lambda i: (0, 0)),
        ],
        out_specs=[
            pl.BlockSpec((BB, F), lambda i: (i, 0)),
            pl.BlockSpec((BB, D), lambda i: (i, 0)),
        ],
        out_shape=[
            jax.ShapeDtypeStruct((B, F), jnp.float32),
            jax.ShapeDtypeStruct((B, D), jnp.float32),
        ],
    )(pre, *ms, W_dec16, b_dec2)

    return (recon, sparse)
